# Initial kernel scaffold; baseline (speedup 1.0000x reference)
import jax
import jax.numpy as jnp
from jax.experimental import pallas as pl


def kernel(x, edge_index, W1_0, b1_0, W2_0, b2_0, W1_1, b1_1, W2_1, b2_1):
    raise NotImplementedError("write your pallas kernel here")


# speedup vs baseline: 6.4823x; 6.4823x over previous
import functools

import jax
import jax.numpy as jnp
from jax import lax
from jax.experimental import pallas as pl
from jax.experimental.pallas import tpu as pltpu
from jax.experimental.pallas import tpu_sc as plsc

N_NODES = 10000
N_EDGES = 320000
D_FEAT = 128

NC = 2
NS = 16
NW = NC * NS
E_PER_W = N_EDGES // NW
K = 128
NCH = E_PER_W // K
TAIL = E_PER_W - NCH * K
N_PAD = 10240
ROWS_PER_TILE = N_PAD // NS
ZR = 128


def _make_sc_agg():
    mesh = plsc.VectorSubcoreMesh(core_axis_name="c", subcore_axis_name="s")

    @functools.partial(
        pl.kernel,
        out_type=jax.ShapeDtypeStruct((NC, N_PAD, D_FEAT), jnp.float32),
        mesh=mesh,
        scratch_types=[
            pltpu.VMEM((K,), jnp.int32),
            pltpu.VMEM((K,), jnp.int32),
            pltpu.VMEM((K, D_FEAT), jnp.float32),
            pltpu.VMEM((TAIL,), jnp.int32),
            pltpu.VMEM((TAIL,), jnp.int32),
            pltpu.VMEM((TAIL, D_FEAT), jnp.float32),
            pltpu.VMEM((ZR, D_FEAT), jnp.float32),
            pltpu.VMEM_SHARED((N_PAD, D_FEAT), jnp.float32),
            pltpu.SemaphoreType.DMA,
        ],
    )
    def sc_agg(h_hbm, src_hbm, dst_hbm, out_hbm,
               srcv, dstv, rows, tsrcv, tdstv, trows, zv, agg_sh, sem):
        cid = lax.axis_index("c")
        sid = lax.axis_index("s")
        wid = sid * NC + cid

        zero16 = jnp.zeros((16,), jnp.float32)

        @pl.loop(0, ZR)
        def _(r):
            @pl.loop(0, D_FEAT // 16)
            def _(j):
                zv[r, pl.ds(j * 16, 16)] = zero16

        rbase = sid * ROWS_PER_TILE

        @pl.loop(0, ROWS_PER_TILE // ZR)
        def _(j):
            pltpu.sync_copy(zv, agg_sh.at[pl.ds(rbase + j * ZR, ZR)])

        plsc.subcore_barrier()

        ebase = wid * E_PER_W

        @pl.loop(0, NCH)
        def _(c):
            off = ebase + c * K
            pltpu.sync_copy(src_hbm.at[pl.ds(off, K)], srcv)
            pltpu.async_copy(h_hbm.at[srcv], rows, sem).wait()
            pltpu.sync_copy(dst_hbm.at[pl.ds(off, K)], dstv)
            pltpu.sync_copy(rows, agg_sh.at[dstv], add=True)

        toff = ebase + NCH * K
        pltpu.sync_copy(src_hbm.at[pl.ds(toff, TAIL)], tsrcv)
        pltpu.async_copy(h_hbm.at[tsrcv], trows, sem).wait()
        pltpu.sync_copy(dst_hbm.at[pl.ds(toff, TAIL)], tdstv)
        pltpu.sync_copy(trows, agg_sh.at[tdstv], add=True)

        plsc.subcore_barrier()

        pltpu.sync_copy(agg_sh.at[pl.ds(rbase, ROWS_PER_TILE)],
                        out_hbm.at[cid].at[pl.ds(rbase, ROWS_PER_TILE)])

    return sc_agg


_sc_agg = _make_sc_agg()


def _mlp_body(h_ref, a_ref, w1_ref, b1_ref, w2_ref, b2_ref, o_ref):
    z = h_ref[...] + a_ref[0] + a_ref[1]
    z1 = jnp.maximum(
        jnp.dot(z, w1_ref[...], preferred_element_type=jnp.float32) + b1_ref[...],
        0.0)
    o_ref[...] = (
        jnp.dot(z1, w2_ref[...], preferred_element_type=jnp.float32) + b2_ref[...])


_MLP_BLK = 1000


def _mlp(h, agg, W1, b1, W2, b2):
    return pl.pallas_call(
        _mlp_body,
        grid=(N_NODES // _MLP_BLK,),
        in_specs=[
            pl.BlockSpec((_MLP_BLK, D_FEAT), lambda i: (i, 0)),
            pl.BlockSpec((NC, _MLP_BLK, D_FEAT), lambda i: (0, i, 0)),
            pl.BlockSpec((D_FEAT, D_FEAT), lambda i: (0, 0)),
            pl.BlockSpec((1, D_FEAT), lambda i: (0, 0)),
            pl.BlockSpec((D_FEAT, D_FEAT), lambda i: (0, 0)),
            pl.BlockSpec((1, D_FEAT), lambda i: (0, 0)),
        ],
        out_specs=pl.BlockSpec((_MLP_BLK, D_FEAT), lambda i: (i, 0)),
        out_shape=jax.ShapeDtypeStruct((N_NODES, D_FEAT), jnp.float32),
    )(h, agg, W1, b1.reshape(1, D_FEAT), W2, b2.reshape(1, D_FEAT))


@jax.jit
def kernel(x, edge_index, W1_0, b1_0, W2_0, b2_0, W1_1, b1_1, W2_1, b2_1):
    src = edge_index[0].astype(jnp.int32)
    dst = edge_index[1].astype(jnp.int32)
    h = x
    for (W1, b1, W2, b2) in ((W1_0, b1_0, W2_0, b2_0), (W1_1, b1_1, W2_1, b2_1)):
        agg = _sc_agg(h, src, dst)
        h = _mlp(h, agg, W1, b1, W2, b2)
    return h


# speedup vs baseline: 11.6024x; 1.7898x over previous
import functools

import jax
import jax.numpy as jnp
from jax import lax
from jax.experimental import pallas as pl
from jax.experimental.pallas import tpu as pltpu
from jax.experimental.pallas import tpu_sc as plsc

N_NODES = 10000
N_EDGES = 320000
D_FEAT = 128

NC = 2
NS = 16
NW = NC * NS
K = 128
NCH = 78
NIT = NCH // 2
E_MAIN = NW * NCH * K
TAIL = (N_EDGES - E_MAIN) // NW
N_PAD = 10240
ROWS_PER_TILE = N_PAD // NS
ZR = 16


def _make_sc_agg():
    mesh = plsc.VectorSubcoreMesh(core_axis_name="c", subcore_axis_name="s")

    @functools.partial(
        pl.kernel,
        out_type=jax.ShapeDtypeStruct((NC, N_PAD, D_FEAT), jnp.float32),
        mesh=mesh,
        scratch_types=[
            pltpu.VMEM((2, 2, K), jnp.int32),
            pltpu.VMEM((2, 2, K), jnp.int32),
            pltpu.VMEM((K, D_FEAT), jnp.float32),
            pltpu.VMEM((K, D_FEAT), jnp.float32),
            pltpu.VMEM((TAIL,), jnp.int32),
            pltpu.VMEM((TAIL,), jnp.int32),
            pltpu.VMEM((TAIL, D_FEAT), jnp.float32),
            pltpu.VMEM((ZR, D_FEAT), jnp.float32),
            pltpu.VMEM_SHARED((N_PAD, D_FEAT), jnp.float32),
            pltpu.SemaphoreType.DMA,
            pltpu.SemaphoreType.DMA,
            pltpu.SemaphoreType.DMA,
        ],
    )
    def sc_agg(h_hbm, src4_hbm, dst4_hbm, srct_hbm, dstt_hbm, out_hbm,
               sidx, didx, rows_a, rows_b, tsrc, tdst, trows, zv, agg_sh,
               gsem, ssem, isem):
        cid = lax.axis_index("c")
        sid = lax.axis_index("s")
        wid = sid * NC + cid

        pltpu.async_copy(src4_hbm.at[wid, 1], sidx.at[1], isem)
        pltpu.async_copy(dst4_hbm.at[wid, 1], didx.at[1], isem)
        pltpu.sync_copy(src4_hbm.at[wid, 0], sidx.at[0])
        pltpu.sync_copy(dst4_hbm.at[wid, 0], didx.at[0])
        pltpu.async_copy(h_hbm.at[sidx.at[0, 0]], rows_a, gsem)
        pltpu.async_copy(h_hbm.at[sidx.at[0, 1]], rows_b, gsem)
        pltpu.sync_copy(srct_hbm.at[wid], tsrc)
        pltpu.sync_copy(dstt_hbm.at[wid], tdst)

        zero16 = jnp.zeros((16,), jnp.float32)

        @pl.loop(0, ZR)
        def _(r):
            @pl.loop(0, D_FEAT // 16)
            def _(j):
                zv[r, pl.ds(j * 16, 16)] = zero16

        rbase = sid * ROWS_PER_TILE

        @pl.loop(0, ROWS_PER_TILE // ZR)
        def _(j):
            pltpu.sync_copy(zv, agg_sh.at[pl.ds(rbase + j * ZR, ZR)])

        plsc.subcore_barrier()

        @pl.loop(0, NIT)
        def _(it):
            p = lax.rem(it, 2)
            pn = lax.rem(it + 1, 2)

            pltpu.make_async_copy(h_hbm.at[sidx.at[p, 0]], rows_a, gsem).wait()
            s0 = pltpu.async_copy(rows_a, agg_sh.at[didx.at[p, 0]], ssem,
                                  add=True)
            pltpu.make_async_copy(h_hbm.at[sidx.at[p, 1]], rows_b, gsem).wait()
            s1 = pltpu.async_copy(rows_b, agg_sh.at[didx.at[p, 1]], ssem,
                                  add=True)

            @pl.when(it + 1 < NIT)
            def _():
                pltpu.make_async_copy(
                    src4_hbm.at[wid, it + 1], sidx.at[pn], isem).wait()
                pltpu.make_async_copy(
                    dst4_hbm.at[wid, it + 1], didx.at[pn], isem).wait()

            s0.wait()

            @pl.when(it + 1 < NIT)
            def _():
                pltpu.async_copy(h_hbm.at[sidx.at[pn, 0]], rows_a, gsem)

            s1.wait()

            @pl.when(it + 1 < NIT)
            def _():
                pltpu.async_copy(h_hbm.at[sidx.at[pn, 1]], rows_b, gsem)

            @pl.when(it + 2 < NIT)
            def _():
                pltpu.async_copy(src4_hbm.at[wid, it + 2], sidx.at[p], isem)
                pltpu.async_copy(dst4_hbm.at[wid, it + 2], didx.at[p], isem)

        pltpu.async_copy(h_hbm.at[tsrc], trows, gsem).wait()
        pltpu.sync_copy(trows, agg_sh.at[tdst], add=True)

        plsc.subcore_barrier()

        pltpu.sync_copy(agg_sh.at[pl.ds(rbase, ROWS_PER_TILE)],
                        out_hbm.at[cid].at[pl.ds(rbase, ROWS_PER_TILE)])

    return sc_agg


_sc_agg = _make_sc_agg()


def _mlp_body(h_ref, a_ref, w1_ref, b1_ref, w2_ref, b2_ref, o_ref):
    z = h_ref[...] + a_ref[0] + a_ref[1]
    z1 = jnp.maximum(
        jnp.dot(z, w1_ref[...], preferred_element_type=jnp.float32) + b1_ref[...],
        0.0)
    o_ref[...] = (
        jnp.dot(z1, w2_ref[...], preferred_element_type=jnp.float32) + b2_ref[...])


_MLP_BLK = 1000


def _mlp(h, agg, W1, b1, W2, b2):
    return pl.pallas_call(
        _mlp_body,
        grid=(N_NODES // _MLP_BLK,),
        in_specs=[
            pl.BlockSpec((_MLP_BLK, D_FEAT), lambda i: (i, 0)),
            pl.BlockSpec((NC, _MLP_BLK, D_FEAT), lambda i: (0, i, 0)),
            pl.BlockSpec((D_FEAT, D_FEAT), lambda i: (0, 0)),
            pl.BlockSpec((1, D_FEAT), lambda i: (0, 0)),
            pl.BlockSpec((D_FEAT, D_FEAT), lambda i: (0, 0)),
            pl.BlockSpec((1, D_FEAT), lambda i: (0, 0)),
        ],
        out_specs=pl.BlockSpec((_MLP_BLK, D_FEAT), lambda i: (i, 0)),
        out_shape=jax.ShapeDtypeStruct((N_NODES, D_FEAT), jnp.float32),
    )(h, agg, W1, b1.reshape(1, D_FEAT), W2, b2.reshape(1, D_FEAT))


@jax.jit
def kernel(x, edge_index, W1_0, b1_0, W2_0, b2_0, W1_1, b1_1, W2_1, b2_1):
    src = edge_index[0].astype(jnp.int32)
    dst = edge_index[1].astype(jnp.int32)
    src4 = src[:E_MAIN].reshape(NW, NIT, 2, K)
    dst4 = dst[:E_MAIN].reshape(NW, NIT, 2, K)
    srct = src[E_MAIN:].reshape(NW, TAIL)
    dstt = dst[E_MAIN:].reshape(NW, TAIL)
    h = x
    for (W1, b1, W2, b2) in ((W1_0, b1_0, W2_0, b2_0), (W1_1, b1_1, W2_1, b2_1)):
        agg = _sc_agg(h, src4, dst4, srct, dstt)
        h = _mlp(h, agg, W1, b1, W2, b2)
    return h


# speedup vs baseline: 12.5444x; 1.0812x over previous
import functools

import jax
import jax.numpy as jnp
from jax import lax
from jax.experimental import pallas as pl
from jax.experimental.pallas import tpu as pltpu
from jax.experimental.pallas import tpu_sc as plsc

N_NODES = 10000
N_EDGES = 320000
D_FEAT = 128

NC = 2
NS = 16
NW = NC * NS
K = 104
NCH = 96
NIT = NCH // 3
E_MAIN = NW * NCH * K
TAIL = (N_EDGES - E_MAIN) // NW
N_PAD = 10240
ROWS_PER_TILE = N_PAD // NS
ZR = 16


def _make_sc_agg():
    mesh = plsc.VectorSubcoreMesh(core_axis_name="c", subcore_axis_name="s")

    @functools.partial(
        pl.kernel,
        out_type=jax.ShapeDtypeStruct((NC, N_PAD, D_FEAT), jnp.float32),
        mesh=mesh,
        scratch_types=[
            pltpu.VMEM((3, 3, K), jnp.int32),
            pltpu.VMEM((3, 3, K), jnp.int32),
            pltpu.VMEM((3, K, D_FEAT), jnp.float32),
            pltpu.VMEM((TAIL,), jnp.int32),
            pltpu.VMEM((TAIL,), jnp.int32),
            pltpu.VMEM((TAIL, D_FEAT), jnp.float32),
            pltpu.VMEM((ZR, D_FEAT), jnp.float32),
            pltpu.VMEM_SHARED((N_PAD, D_FEAT), jnp.float32),
            pltpu.SemaphoreType.DMA,
            pltpu.SemaphoreType.DMA,
            pltpu.SemaphoreType.DMA,
        ],
    )
    def sc_agg(h_hbm, src4_hbm, dst4_hbm, srct_hbm, dstt_hbm, out_hbm,
               sidx, didx, rows, tsrc, tdst, trows, zv, agg_sh,
               gsem, ssem, isem):
        cid = lax.axis_index("c")
        sid = lax.axis_index("s")
        wid = sid * NC + cid

        pltpu.async_copy(src4_hbm.at[wid, 1], sidx.at[1], isem)
        pltpu.async_copy(dst4_hbm.at[wid, 1], didx.at[1], isem)
        pltpu.sync_copy(src4_hbm.at[wid, 0], sidx.at[0])
        pltpu.sync_copy(dst4_hbm.at[wid, 0], didx.at[0])
        pltpu.async_copy(h_hbm.at[sidx.at[0, 0]], rows.at[0], gsem)
        pltpu.async_copy(h_hbm.at[sidx.at[0, 1]], rows.at[1], gsem)
        pltpu.sync_copy(srct_hbm.at[wid], tsrc)
        pltpu.sync_copy(dstt_hbm.at[wid], tdst)

        zero16 = jnp.zeros((16,), jnp.float32)

        @pl.loop(0, ZR)
        def _(r):
            @pl.loop(0, D_FEAT // 16)
            def _(j):
                zv[r, pl.ds(j * 16, 16)] = zero16

        rbase = sid * ROWS_PER_TILE

        @pl.loop(0, ROWS_PER_TILE // ZR)
        def _(j):
            pltpu.sync_copy(zv, agg_sh.at[pl.ds(rbase + j * ZR, ZR)])

        plsc.subcore_barrier()

        @pl.loop(0, NIT)
        def _(it):
            p = lax.rem(it, 3)
            pn = lax.rem(it + 1, 3)
            pr = lax.rem(it + 2, 3)
            scatters = []
            for b in range(3):
                c = 3 * it + b

                pltpu.make_async_copy(
                    h_hbm.at[sidx.at[p, b]], rows.at[b], gsem).wait()
                scatters.append(pltpu.async_copy(
                    rows.at[b], agg_sh.at[didx.at[p, b]], ssem, add=True))

                if b == 0:
                    @pl.when(it >= 1)
                    def _():
                        pltpu.make_async_copy(
                            rows.at[2], agg_sh.at[didx.at[p, 0]], ssem).wait()

                    @pl.when(c + 2 < NCH)
                    def _():
                        pltpu.async_copy(
                            h_hbm.at[sidx.at[p, 2]], rows.at[2], gsem)
                elif b == 1:
                    scatters[0].wait()

                    @pl.when(it + 1 < NIT)
                    def _():
                        pltpu.make_async_copy(
                            src4_hbm.at[wid, it + 1], sidx.at[pn], isem).wait()
                        pltpu.make_async_copy(
                            dst4_hbm.at[wid, it + 1], didx.at[pn], isem).wait()
                        pltpu.async_copy(
                            h_hbm.at[sidx.at[pn, 0]], rows.at[0], gsem)

                    @pl.when(it + 2 < NIT)
                    def _():
                        pltpu.async_copy(
                            src4_hbm.at[wid, it + 2], sidx.at[pr], isem)
                        pltpu.async_copy(
                            dst4_hbm.at[wid, it + 2], didx.at[pr], isem)
                else:
                    scatters[1].wait()

                    @pl.when(it + 1 < NIT)
                    def _():
                        pltpu.async_copy(
                            h_hbm.at[sidx.at[pn, 1]], rows.at[1], gsem)

        pltpu.make_async_copy(
            rows.at[2], agg_sh.at[didx.at[0, 0]], ssem).wait()
        pltpu.async_copy(h_hbm.at[tsrc], trows, gsem).wait()
        pltpu.sync_copy(trows, agg_sh.at[tdst], add=True)

        plsc.subcore_barrier()

        pltpu.sync_copy(agg_sh.at[pl.ds(rbase, ROWS_PER_TILE)],
                        out_hbm.at[cid].at[pl.ds(rbase, ROWS_PER_TILE)])

    return sc_agg


_sc_agg = _make_sc_agg()


def _mlp_body(h_ref, a_ref, w1_ref, b1_ref, w2_ref, b2_ref, o_ref):
    z = h_ref[...] + a_ref[0] + a_ref[1]
    z1 = jnp.maximum(
        jnp.dot(z, w1_ref[...], preferred_element_type=jnp.float32) + b1_ref[...],
        0.0)
    o_ref[...] = (
        jnp.dot(z1, w2_ref[...], preferred_element_type=jnp.float32) + b2_ref[...])


_MLP_BLK = 1000


def _mlp(h, agg, W1, b1, W2, b2):
    return pl.pallas_call(
        _mlp_body,
        grid=(N_NODES // _MLP_BLK,),
        in_specs=[
            pl.BlockSpec((_MLP_BLK, D_FEAT), lambda i: (i, 0)),
            pl.BlockSpec((NC, _MLP_BLK, D_FEAT), lambda i: (0, i, 0)),
            pl.BlockSpec((D_FEAT, D_FEAT), lambda i: (0, 0)),
            pl.BlockSpec((1, D_FEAT), lambda i: (0, 0)),
            pl.BlockSpec((D_FEAT, D_FEAT), lambda i: (0, 0)),
            pl.BlockSpec((1, D_FEAT), lambda i: (0, 0)),
        ],
        out_specs=pl.BlockSpec((_MLP_BLK, D_FEAT), lambda i: (i, 0)),
        out_shape=jax.ShapeDtypeStruct((N_NODES, D_FEAT), jnp.float32),
    )(h, agg, W1, b1.reshape(1, D_FEAT), W2, b2.reshape(1, D_FEAT))


@jax.jit
def kernel(x, edge_index, W1_0, b1_0, W2_0, b2_0, W1_1, b1_1, W2_1, b2_1):
    src = edge_index[0].astype(jnp.int32)
    dst = edge_index[1].astype(jnp.int32)
    src4 = src[:E_MAIN].reshape(NW, NIT, 3, K)
    dst4 = dst[:E_MAIN].reshape(NW, NIT, 3, K)
    srct = src[E_MAIN:].reshape(NW, TAIL)
    dstt = dst[E_MAIN:].reshape(NW, TAIL)
    h = x
    for (W1, b1, W2, b2) in ((W1_0, b1_0, W2_0, b2_0), (W1_1, b1_1, W2_1, b2_1)):
        agg = _sc_agg(h, src4, dst4, srct, dstt)
        h = _mlp(h, agg, W1, b1, W2, b2)
    return h


# speedup vs baseline: 14.0618x; 1.1210x over previous
import functools

import jax
import jax.numpy as jnp
import numpy as np
from jax import lax
from jax.experimental import pallas as pl
from jax.experimental.pallas import tpu as pltpu
from jax.experimental.pallas import tpu_sc as plsc

N_NODES = 10000
N_EDGES = 320000
D_FEAT = 128

NC = 2
NS = 16
NW = NC * NS
K = 104
NCH = 96
NIT = NCH // 3
E_PER_W = NCH * K
E_MAIN = NW * E_PER_W
TAIL = (N_EDGES - E_MAIN) // NW
NIDX = 6
N_PAD = 10240
ROWS_PER_TILE = N_PAD // NS

_ZERO_ROWS = np.zeros((ROWS_PER_TILE, D_FEAT), np.float32)


def _make_sc_agg():
    mesh = plsc.VectorSubcoreMesh(core_axis_name="c", subcore_axis_name="s")

    @functools.partial(
        pl.kernel,
        out_type=jax.ShapeDtypeStruct((NC, N_PAD, D_FEAT), jnp.float32),
        mesh=mesh,
        scratch_types=[
            pltpu.VMEM((NIDX, K), jnp.int32),
            pltpu.VMEM((NIDX, K), jnp.int32),
            pltpu.VMEM((3, K, D_FEAT), jnp.float32),
            pltpu.VMEM((TAIL,), jnp.int32),
            pltpu.VMEM((TAIL,), jnp.int32),
            pltpu.VMEM((TAIL, D_FEAT), jnp.float32),
            pltpu.VMEM_SHARED((N_PAD, D_FEAT), jnp.float32),
            pltpu.SemaphoreType.DMA,
            pltpu.SemaphoreType.DMA,
            pltpu.SemaphoreType.DMA,
        ],
    )
    def sc_agg(h_hbm, src_hbm, dst_hbm, zeros_hbm, out_hbm,
               sidx, didx, rows, tsrc, tdst, trows, agg_sh,
               gsem, ssem, isem):
        cid = lax.axis_index("c")
        sid = lax.axis_index("s")
        wid = sid * NC + cid
        ebase = wid * E_PER_W

        def idx_refill(c, slot):
            pltpu.async_copy(src_hbm.at[pl.ds(ebase + c * K, K)],
                             sidx.at[slot], isem)
            pltpu.async_copy(dst_hbm.at[pl.ds(ebase + c * K, K)],
                             didx.at[slot], isem)

        def idx_wait(c, slot):
            pltpu.make_async_copy(src_hbm.at[pl.ds(ebase + c * K, K)],
                                  sidx.at[slot], isem).wait()
            pltpu.make_async_copy(dst_hbm.at[pl.ds(ebase + c * K, K)],
                                  didx.at[slot], isem).wait()

        for c in range(5):
            idx_refill(c, c)
        idx_wait(0, 0)
        idx_wait(1, 1)

        pltpu.async_copy(h_hbm.at[sidx.at[0]], rows.at[0], gsem)
        pltpu.async_copy(h_hbm.at[sidx.at[1]], rows.at[1], gsem)
        pltpu.sync_copy(src_hbm.at[pl.ds(E_MAIN + wid * TAIL, TAIL)], tsrc)
        pltpu.sync_copy(dst_hbm.at[pl.ds(E_MAIN + wid * TAIL, TAIL)], tdst)

        rbase = sid * ROWS_PER_TILE
        pltpu.sync_copy(zeros_hbm, agg_sh.at[pl.ds(rbase, ROWS_PER_TILE)])

        plsc.subcore_barrier()

        @pl.loop(0, NIT)
        def _(it):
            c0 = 3 * it
            scatters = []
            for b in range(3):
                c = c0 + b
                q = lax.rem(c, NIDX)
                q2 = lax.rem(c + 2, NIDX)

                pltpu.make_async_copy(
                    h_hbm.at[sidx.at[q]], rows.at[b], gsem).wait()
                scatters.append(pltpu.async_copy(
                    rows.at[b], agg_sh.at[didx.at[q]], ssem, add=True))

                if b == 0:
                    @pl.when(it >= 1)
                    def _():
                        pltpu.make_async_copy(
                            rows.at[2], agg_sh.at[didx.at[q]], ssem).wait()
                else:
                    scatters[b - 1].wait()

                @pl.when(c + 5 < NCH)
                def _():
                    idx_refill(c + 5, lax.rem(c + 5, NIDX))

                @pl.when(c + 2 < NCH)
                def _():
                    idx_wait(c + 2, q2)
                    pltpu.async_copy(
                        h_hbm.at[sidx.at[q2]], rows.at[(b + 2) % 3], gsem)

        pltpu.make_async_copy(
            rows.at[(NCH - 1) % 3], agg_sh.at[didx.at[0]], ssem).wait()
        pltpu.async_copy(h_hbm.at[tsrc], trows, gsem).wait()
        pltpu.sync_copy(trows, agg_sh.at[tdst], add=True)

        plsc.subcore_barrier()

        pltpu.sync_copy(agg_sh.at[pl.ds(rbase, ROWS_PER_TILE)],
                        out_hbm.at[cid].at[pl.ds(rbase, ROWS_PER_TILE)])

    return sc_agg


_sc_agg = _make_sc_agg()


def _mlp_body(h_ref, a_ref, w1_ref, b1_ref, w2_ref, b2_ref, o_ref):
    z = h_ref[...] + a_ref[0] + a_ref[1]
    z1 = jnp.maximum(
        jnp.dot(z, w1_ref[...], preferred_element_type=jnp.float32) + b1_ref[...],
        0.0)
    o_ref[...] = (
        jnp.dot(z1, w2_ref[...], preferred_element_type=jnp.float32) + b2_ref[...])


_MLP_BLK = 1000


def _mlp(h, agg, W1, b1, W2, b2):
    return pl.pallas_call(
        _mlp_body,
        grid=(N_NODES // _MLP_BLK,),
        in_specs=[
            pl.BlockSpec((_MLP_BLK, D_FEAT), lambda i: (i, 0)),
            pl.BlockSpec((NC, _MLP_BLK, D_FEAT), lambda i: (0, i, 0)),
            pl.BlockSpec((D_FEAT, D_FEAT), lambda i: (0, 0)),
            pl.BlockSpec((1, D_FEAT), lambda i: (0, 0)),
            pl.BlockSpec((D_FEAT, D_FEAT), lambda i: (0, 0)),
            pl.BlockSpec((1, D_FEAT), lambda i: (0, 0)),
        ],
        out_specs=pl.BlockSpec((_MLP_BLK, D_FEAT), lambda i: (i, 0)),
        out_shape=jax.ShapeDtypeStruct((N_NODES, D_FEAT), jnp.float32),
    )(h, agg, W1, b1.reshape(1, D_FEAT), W2, b2.reshape(1, D_FEAT))


@jax.jit
def kernel(x, edge_index, W1_0, b1_0, W2_0, b2_0, W1_1, b1_1, W2_1, b2_1):
    src = edge_index[0].astype(jnp.int32)
    dst = edge_index[1].astype(jnp.int32)
    h = x
    for (W1, b1, W2, b2) in ((W1_0, b1_0, W2_0, b2_0), (W1_1, b1_1, W2_1, b2_1)):
        agg = _sc_agg(h, src, dst, _ZERO_ROWS)
        h = _mlp(h, agg, W1, b1, W2, b2)
    return h


# speedup vs baseline: 14.8998x; 1.0596x over previous
import functools

import jax
import jax.numpy as jnp
import numpy as np
from jax import lax
from jax.experimental import pallas as pl
from jax.experimental.pallas import tpu as pltpu
from jax.experimental.pallas import tpu_sc as plsc

N_NODES = 10000
N_EDGES = 320000
D_FEAT = 128

NC = 2
NS = 16
NW = NC * NS
K = 128
NCH = 78
NIT = NCH // 3
E_PER_W = NCH * K
E_MAIN = NW * E_PER_W
TAIL = (N_EDGES - E_MAIN) // NW
NIDX = 5
RPT = 624
RPT_LAST = N_NODES - 15 * RPT

_ZERO_ROWS = np.zeros((RPT_LAST, D_FEAT), np.float32)


def _make_sc_agg():
    mesh = plsc.VectorSubcoreMesh(core_axis_name="c", subcore_axis_name="s")

    @functools.partial(
        pl.kernel,
        out_type=jax.ShapeDtypeStruct((NC, N_NODES, D_FEAT), jnp.float32),
        mesh=mesh,
        scratch_types=[
            pltpu.VMEM((NIDX, 2, K), jnp.int32),
            pltpu.VMEM((3, K, D_FEAT), jnp.float32),
            pltpu.VMEM_SHARED((N_NODES, D_FEAT), jnp.float32),
            pltpu.SemaphoreType.DMA,
            pltpu.SemaphoreType.DMA,
            pltpu.SemaphoreType.DMA,
        ],
    )
    def sc_agg(h_hbm, ei_hbm, zeros_hbm, out_hbm,
               eidx, rows, agg_sh, gsem, ssem, isem):
        cid = lax.axis_index("c")
        sid = lax.axis_index("s")
        wid = sid * NC + cid
        ebase = wid * E_PER_W

        def idx_refill(c, slot):
            pltpu.async_copy(
                ei_hbm.at[pl.ds(0, 2), pl.ds(ebase + c * K, K)],
                eidx.at[slot], isem)

        def idx_wait(c, slot):
            pltpu.make_async_copy(
                ei_hbm.at[pl.ds(0, 2), pl.ds(ebase + c * K, K)],
                eidx.at[slot], isem).wait()

        for c in range(4):
            idx_refill(c, c)
        idx_wait(0, 0)
        idx_wait(1, 1)

        pltpu.async_copy(h_hbm.at[eidx.at[0, 0]], rows.at[0], gsem)
        pltpu.async_copy(h_hbm.at[eidx.at[1, 0]], rows.at[1], gsem)

        rbase = sid * RPT
        pltpu.sync_copy(zeros_hbm.at[pl.ds(0, RPT)],
                        agg_sh.at[pl.ds(rbase, RPT)])

        @pl.when(sid == NS - 1)
        def _():
            pltpu.sync_copy(zeros_hbm.at[pl.ds(0, RPT_LAST - RPT)],
                            agg_sh.at[pl.ds(15 * RPT + RPT, RPT_LAST - RPT)])

        plsc.subcore_barrier()

        @pl.loop(0, NIT)
        def _(it):
            c0 = 3 * it
            scatters = []
            for b in range(3):
                c = c0 + b
                q = lax.rem(c, NIDX)
                q2 = lax.rem(c + 2, NIDX)

                pltpu.make_async_copy(
                    h_hbm.at[eidx.at[q, 0]], rows.at[b], gsem).wait()
                scatters.append(pltpu.async_copy(
                    rows.at[b], agg_sh.at[eidx.at[q, 1]], ssem, add=True))

                if b == 0:
                    @pl.when(it >= 1)
                    def _():
                        pltpu.make_async_copy(
                            rows.at[2], agg_sh.at[eidx.at[q, 1]], ssem).wait()
                else:
                    scatters[b - 1].wait()

                @pl.when(c + 4 < NCH)
                def _():
                    idx_refill(c + 4, lax.rem(c + 4, NIDX))

                @pl.when(c + 2 < NCH)
                def _():
                    idx_wait(c + 2, q2)
                    pltpu.async_copy(
                        h_hbm.at[eidx.at[q2, 0]], rows.at[(b + 2) % 3], gsem)

        pltpu.make_async_copy(
            rows.at[2], agg_sh.at[eidx.at[0, 1]], ssem).wait()

        @pl.when(wid < 4)
        def _():
            pltpu.sync_copy(
                ei_hbm.at[pl.ds(0, 2), pl.ds(E_MAIN + wid * K, K)],
                eidx.at[0])
            pltpu.async_copy(
                h_hbm.at[eidx.at[0, 0]], rows.at[0], gsem).wait()
            pltpu.sync_copy(rows.at[0], agg_sh.at[eidx.at[0, 1]], add=True)

        plsc.subcore_barrier()

        pltpu.sync_copy(agg_sh.at[pl.ds(rbase, RPT)],
                        out_hbm.at[cid].at[pl.ds(rbase, RPT)])

        @pl.when(sid == NS - 1)
        def _():
            pltpu.sync_copy(
                agg_sh.at[pl.ds(16 * RPT, RPT_LAST - RPT)],
                out_hbm.at[cid].at[pl.ds(16 * RPT, RPT_LAST - RPT)])

    return sc_agg


_sc_agg = _make_sc_agg()


def _mlp_body(h_ref, a_ref, w1_ref, b1_ref, w2_ref, b2_ref, o_ref):
    z = h_ref[...] + a_ref[0] + a_ref[1]
    z1 = jnp.maximum(
        jnp.dot(z, w1_ref[...], preferred_element_type=jnp.float32) + b1_ref[...],
        0.0)
    o_ref[...] = (
        jnp.dot(z1, w2_ref[...], preferred_element_type=jnp.float32) + b2_ref[...])


_MLP_BLK = 1000


def _mlp(h, agg, W1, b1, W2, b2):
    return pl.pallas_call(
        _mlp_body,
        grid=(N_NODES // _MLP_BLK,),
        in_specs=[
            pl.BlockSpec((_MLP_BLK, D_FEAT), lambda i: (i, 0)),
            pl.BlockSpec((NC, _MLP_BLK, D_FEAT), lambda i: (0, i, 0)),
            pl.BlockSpec((D_FEAT, D_FEAT), lambda i: (0, 0)),
            pl.BlockSpec((1, D_FEAT), lambda i: (0, 0)),
            pl.BlockSpec((D_FEAT, D_FEAT), lambda i: (0, 0)),
            pl.BlockSpec((1, D_FEAT), lambda i: (0, 0)),
        ],
        out_specs=pl.BlockSpec((_MLP_BLK, D_FEAT), lambda i: (i, 0)),
        out_shape=jax.ShapeDtypeStruct((N_NODES, D_FEAT), jnp.float32),
    )(h, agg, W1, b1.reshape(1, D_FEAT), W2, b2.reshape(1, D_FEAT))


@jax.jit
def kernel(x, edge_index, W1_0, b1_0, W2_0, b2_0, W1_1, b1_1, W2_1, b2_1):
    ei = edge_index.astype(jnp.int32)
    h = x
    for (W1, b1, W2, b2) in ((W1_0, b1_0, W2_0, b2_0), (W1_1, b1_1, W2_1, b2_1)):
        agg = _sc_agg(h, ei, _ZERO_ROWS)
        h = _mlp(h, agg, W1, b1, W2, b2)
    return h


# speedup vs baseline: 15.6061x; 1.0474x over previous
import functools

import jax
import jax.numpy as jnp
import numpy as np
from jax import lax
from jax.experimental import pallas as pl
from jax.experimental.pallas import tpu as pltpu
from jax.experimental.pallas import tpu_sc as plsc

N_NODES = 10000
N_EDGES = 320000
D_FEAT = 128

NC = 2
NS = 16
NW = NC * NS
K = 128
NCH = 78
NIT = NCH // 3
E_PER_W = NCH * K
E_MAIN = NW * E_PER_W
TAIL = (N_EDGES - E_MAIN) // NW
NIDX = 5
RPT = 624
RPT_LAST = N_NODES - 15 * RPT

_ZERO_ROWS = np.zeros((RPT_LAST, D_FEAT), np.float32)


def _make_sc_agg():
    mesh = plsc.VectorSubcoreMesh(core_axis_name="c", subcore_axis_name="s")

    @functools.partial(
        pl.kernel,
        out_type=jax.ShapeDtypeStruct((NC, N_NODES, D_FEAT), jnp.float32),
        mesh=mesh,
        scratch_types=[
            pltpu.VMEM((NIDX, 2, K), jnp.int32),
            pltpu.VMEM((3, K, D_FEAT), jnp.float32),
            pltpu.VMEM_SHARED((N_NODES, D_FEAT), jnp.float32),
            pltpu.SemaphoreType.DMA,
            pltpu.SemaphoreType.DMA,
            pltpu.SemaphoreType.DMA,
        ],
    )
    def sc_agg(h_hbm, ei_hbm, zeros_hbm, out_hbm,
               eidx, rows, agg_sh, gsem, ssem, isem):
        cid = lax.axis_index("c")
        sid = lax.axis_index("s")
        wid = sid * NC + cid
        ebase = wid * E_PER_W

        def idx_refill(c, slot):
            pltpu.async_copy(
                ei_hbm.at[pl.ds(0, 2), pl.ds(ebase + c * K, K)],
                eidx.at[slot], isem)

        def idx_wait(c, slot):
            pltpu.make_async_copy(
                ei_hbm.at[pl.ds(0, 2), pl.ds(ebase + c * K, K)],
                eidx.at[slot], isem).wait()

        for c in range(4):
            idx_refill(c, c)
        idx_wait(0, 0)
        idx_wait(1, 1)

        pltpu.async_copy(h_hbm.at[eidx.at[0, 0]], rows.at[0], gsem)
        pltpu.async_copy(h_hbm.at[eidx.at[1, 0]], rows.at[1], gsem)

        rbase = sid * RPT

        @pl.when(cid == 0)
        def _():
            pltpu.sync_copy(h_hbm.at[pl.ds(rbase, RPT)],
                            agg_sh.at[pl.ds(rbase, RPT)])

            @pl.when(sid == NS - 1)
            def _():
                pltpu.sync_copy(h_hbm.at[pl.ds(16 * RPT, RPT_LAST - RPT)],
                                agg_sh.at[pl.ds(16 * RPT, RPT_LAST - RPT)])

        @pl.when(cid == 1)
        def _():
            pltpu.sync_copy(zeros_hbm.at[pl.ds(0, RPT)],
                            agg_sh.at[pl.ds(rbase, RPT)])

            @pl.when(sid == NS - 1)
            def _():
                pltpu.sync_copy(zeros_hbm.at[pl.ds(0, RPT_LAST - RPT)],
                                agg_sh.at[pl.ds(16 * RPT, RPT_LAST - RPT)])

        plsc.subcore_barrier()

        @pl.loop(0, NIT)
        def _(it):
            c0 = 3 * it
            scatters = []
            for b in range(3):
                c = c0 + b
                q = lax.rem(c, NIDX)
                q2 = lax.rem(c + 2, NIDX)

                pltpu.make_async_copy(
                    h_hbm.at[eidx.at[q, 0]], rows.at[b], gsem).wait()
                scatters.append(pltpu.async_copy(
                    rows.at[b], agg_sh.at[eidx.at[q, 1]], ssem, add=True))

                if b == 0:
                    @pl.when(it >= 1)
                    def _():
                        pltpu.make_async_copy(
                            rows.at[2], agg_sh.at[eidx.at[q, 1]], ssem).wait()
                else:
                    scatters[b - 1].wait()

                @pl.when(c + 4 < NCH)
                def _():
                    idx_refill(c + 4, lax.rem(c + 4, NIDX))

                @pl.when(c + 2 < NCH)
                def _():
                    idx_wait(c + 2, q2)
                    pltpu.async_copy(
                        h_hbm.at[eidx.at[q2, 0]], rows.at[(b + 2) % 3], gsem)

        pltpu.make_async_copy(
            rows.at[2], agg_sh.at[eidx.at[0, 1]], ssem).wait()

        @pl.when(wid < 4)
        def _():
            pltpu.sync_copy(
                ei_hbm.at[pl.ds(0, 2), pl.ds(E_MAIN + wid * K, K)],
                eidx.at[0])
            pltpu.async_copy(
                h_hbm.at[eidx.at[0, 0]], rows.at[0], gsem).wait()
            pltpu.sync_copy(rows.at[0], agg_sh.at[eidx.at[0, 1]], add=True)

        plsc.subcore_barrier()

        pltpu.sync_copy(agg_sh.at[pl.ds(rbase, RPT)],
                        out_hbm.at[cid].at[pl.ds(rbase, RPT)])

        @pl.when(sid == NS - 1)
        def _():
            pltpu.sync_copy(
                agg_sh.at[pl.ds(16 * RPT, RPT_LAST - RPT)],
                out_hbm.at[cid].at[pl.ds(16 * RPT, RPT_LAST - RPT)])

    return sc_agg


_sc_agg = _make_sc_agg()


def _mlp_body(a_ref, w1_ref, b1_ref, w2_ref, b2_ref, o_ref):
    z = a_ref[0] + a_ref[1]
    z1 = jnp.maximum(
        jnp.dot(z, w1_ref[...], preferred_element_type=jnp.float32) + b1_ref[...],
        0.0)
    o_ref[...] = (
        jnp.dot(z1, w2_ref[...], preferred_element_type=jnp.float32) + b2_ref[...])


_MLP_BLK = 2000


def _mlp(agg, W1, b1, W2, b2):
    return pl.pallas_call(
        _mlp_body,
        grid=(N_NODES // _MLP_BLK,),
        in_specs=[
            pl.BlockSpec((NC, _MLP_BLK, D_FEAT), lambda i: (0, i, 0)),
            pl.BlockSpec((D_FEAT, D_FEAT), lambda i: (0, 0)),
            pl.BlockSpec((1, D_FEAT), lambda i: (0, 0)),
            pl.BlockSpec((D_FEAT, D_FEAT), lambda i: (0, 0)),
            pl.BlockSpec((1, D_FEAT), lambda i: (0, 0)),
        ],
        out_specs=pl.BlockSpec((_MLP_BLK, D_FEAT), lambda i: (i, 0)),
        out_shape=jax.ShapeDtypeStruct((N_NODES, D_FEAT), jnp.float32),
    )(agg, W1, b1.reshape(1, D_FEAT), W2, b2.reshape(1, D_FEAT))


@jax.jit
def kernel(x, edge_index, W1_0, b1_0, W2_0, b2_0, W1_1, b1_1, W2_1, b2_1):
    ei = edge_index.astype(jnp.int32)
    h = x
    for (W1, b1, W2, b2) in ((W1_0, b1_0, W2_0, b2_0), (W1_1, b1_1, W2_1, b2_1)):
        agg = _sc_agg(h, ei, _ZERO_ROWS)
        h = _mlp(agg, W1, b1, W2, b2)
    return h


# speedup vs baseline: 15.7535x; 1.0094x over previous
import functools

import jax
import jax.numpy as jnp
import numpy as np
from jax import lax
from jax.experimental import pallas as pl
from jax.experimental.pallas import tpu as pltpu
from jax.experimental.pallas import tpu_sc as plsc

N_NODES = 10000
N_EDGES = 320000
D_FEAT = 128

NC = 2
NS = 16
NW = NC * NS
K = 128
NCH = 78
NIT = NCH // 3
E_PER_W = NCH * K
E_MAIN = NW * E_PER_W
TAIL = (N_EDGES - E_MAIN) // NW
NIDX = 5
RPT = 624
RPT_LAST = N_NODES - 15 * RPT

_ZERO_ROWS = np.zeros((RPT_LAST, D_FEAT), np.float32)


def _make_sc_agg():
    mesh = plsc.VectorSubcoreMesh(core_axis_name="c", subcore_axis_name="s")

    @functools.partial(
        pl.kernel,
        out_type=jax.ShapeDtypeStruct((NC, N_NODES, D_FEAT), jnp.float32),
        mesh=mesh,
        scratch_types=[
            pltpu.VMEM((NIDX, 2, K), jnp.int32),
            pltpu.VMEM((3, K, D_FEAT), jnp.float32),
            pltpu.VMEM_SHARED((N_NODES, D_FEAT), jnp.float32),
            pltpu.SemaphoreType.DMA,
            pltpu.SemaphoreType.DMA,
            pltpu.SemaphoreType.DMA,
        ],
    )
    def sc_agg(h_hbm, ei_hbm, zeros_hbm, out_hbm,
               eidx, rows, agg_sh, gsem, ssem, isem):
        cid = lax.axis_index("c")
        sid = lax.axis_index("s")
        wid = sid * NC + cid
        ebase = wid * E_PER_W

        def idx_refill(c, slot):
            pltpu.async_copy(
                ei_hbm.at[pl.ds(0, 2), pl.ds(ebase + c * K, K)],
                eidx.at[slot], isem)

        def idx_wait(c, slot):
            pltpu.make_async_copy(
                ei_hbm.at[pl.ds(0, 2), pl.ds(ebase + c * K, K)],
                eidx.at[slot], isem).wait()

        for c in range(4):
            idx_refill(c, c)
        idx_wait(0, 0)
        idx_wait(1, 1)

        pltpu.async_copy(h_hbm.at[eidx.at[0, 0]], rows.at[0], gsem)
        pltpu.async_copy(h_hbm.at[eidx.at[1, 0]], rows.at[1], gsem)

        rbase = sid * RPT

        @pl.when(cid == 0)
        def _():
            pltpu.sync_copy(h_hbm.at[pl.ds(rbase, RPT)],
                            agg_sh.at[pl.ds(rbase, RPT)])

            @pl.when(sid == NS - 1)
            def _():
                pltpu.sync_copy(h_hbm.at[pl.ds(16 * RPT, RPT_LAST - RPT)],
                                agg_sh.at[pl.ds(16 * RPT, RPT_LAST - RPT)])

        @pl.when(cid == 1)
        def _():
            pltpu.sync_copy(zeros_hbm.at[pl.ds(0, RPT)],
                            agg_sh.at[pl.ds(rbase, RPT)])

            @pl.when(sid == NS - 1)
            def _():
                pltpu.sync_copy(zeros_hbm.at[pl.ds(0, RPT_LAST - RPT)],
                                agg_sh.at[pl.ds(16 * RPT, RPT_LAST - RPT)])

        plsc.subcore_barrier()

        @pl.loop(0, NIT)
        def _(it):
            c0 = 3 * it
            scatters = []
            for b in range(3):
                c = c0 + b
                q = lax.rem(c, NIDX)
                q2 = lax.rem(c + 2, NIDX)

                pltpu.make_async_copy(
                    h_hbm.at[eidx.at[q, 0]], rows.at[b], gsem).wait()
                scatters.append(pltpu.async_copy(
                    rows.at[b], agg_sh.at[eidx.at[q, 1]], ssem, add=True))

                if b == 0:
                    @pl.when(it >= 1)
                    def _():
                        pltpu.make_async_copy(
                            rows.at[2], agg_sh.at[eidx.at[q, 1]], ssem).wait()
                else:
                    scatters[b - 1].wait()

                @pl.when(c + 4 < NCH)
                def _():
                    idx_refill(c + 4, lax.rem(c + 4, NIDX))

                @pl.when(c + 2 < NCH)
                def _():
                    idx_wait(c + 2, q2)
                    pltpu.async_copy(
                        h_hbm.at[eidx.at[q2, 0]], rows.at[(b + 2) % 3], gsem)

        pltpu.make_async_copy(
            rows.at[2], agg_sh.at[eidx.at[0, 1]], ssem).wait()

        @pl.when(wid < 4)
        def _():
            pltpu.sync_copy(
                ei_hbm.at[pl.ds(0, 2), pl.ds(E_MAIN + wid * K, K)],
                eidx.at[0])
            pltpu.async_copy(
                h_hbm.at[eidx.at[0, 0]], rows.at[0], gsem).wait()
            pltpu.sync_copy(rows.at[0], agg_sh.at[eidx.at[0, 1]], add=True)

        plsc.subcore_barrier()

        pltpu.sync_copy(agg_sh.at[pl.ds(rbase, RPT)],
                        out_hbm.at[cid].at[pl.ds(rbase, RPT)])

        @pl.when(sid == NS - 1)
        def _():
            pltpu.sync_copy(
                agg_sh.at[pl.ds(16 * RPT, RPT_LAST - RPT)],
                out_hbm.at[cid].at[pl.ds(16 * RPT, RPT_LAST - RPT)])

    return sc_agg


_sc_agg = _make_sc_agg()


def _mlp_body(a_ref, w1_ref, b1_ref, w2_ref, b2_ref, o_ref):
    z = a_ref[0] + a_ref[1]
    z1 = jnp.maximum(
        jnp.dot(z, w1_ref[...], preferred_element_type=jnp.float32) + b1_ref[...],
        0.0)
    o_ref[...] = (
        jnp.dot(z1, w2_ref[...], preferred_element_type=jnp.float32) + b2_ref[...])


_MLP_BLK = 5000


def _mlp(agg, W1, b1, W2, b2):
    return pl.pallas_call(
        _mlp_body,
        grid=(N_NODES // _MLP_BLK,),
        in_specs=[
            pl.BlockSpec((NC, _MLP_BLK, D_FEAT), lambda i: (0, i, 0)),
            pl.BlockSpec((D_FEAT, D_FEAT), lambda i: (0, 0)),
            pl.BlockSpec((1, D_FEAT), lambda i: (0, 0)),
            pl.BlockSpec((D_FEAT, D_FEAT), lambda i: (0, 0)),
            pl.BlockSpec((1, D_FEAT), lambda i: (0, 0)),
        ],
        out_specs=pl.BlockSpec((_MLP_BLK, D_FEAT), lambda i: (i, 0)),
        out_shape=jax.ShapeDtypeStruct((N_NODES, D_FEAT), jnp.float32),
    )(agg, W1, b1.reshape(1, D_FEAT), W2, b2.reshape(1, D_FEAT))


@jax.jit
def kernel(x, edge_index, W1_0, b1_0, W2_0, b2_0, W1_1, b1_1, W2_1, b2_1):
    ei = edge_index.astype(jnp.int32)
    h = x
    for (W1, b1, W2, b2) in ((W1_0, b1_0, W2_0, b2_0), (W1_1, b1_1, W2_1, b2_1)):
        agg = _sc_agg(h, ei, _ZERO_ROWS)
        h = _mlp(agg, W1, b1, W2, b2)
    return h
